# baseline (device time: 121520 ns/iter reference)
import numpy as np
import jax
import jax.numpy as jnp
from jax import lax
from jax.experimental import pallas as pl
from jax.experimental.pallas import tpu as pltpu

N_DEV = 4
SQ = 1024
D = 1024
HQ = 8
DH = 128
CH = SQ // N_DEV
SCALE = 0.08838834764831843

_inv = 1.0 / (10000.0 ** (np.arange(0, DH, 2) / DH))
_pos = np.arange(SQ)[:, None] * _inv[None, :]
_COS = np.tile(np.repeat(np.cos(_pos), 2, axis=-1), (1, HQ)).astype(np.float32)
_SIN = np.tile(np.repeat(np.sin(_pos), 2, axis=-1), (1, HQ)).astype(np.float32)


def kernel(x, Wq, Wk, Wv, Wo):

    def body(x_ref, wq_ref, wk_ref, wv_ref, wo_ref, cos_ref, sin_ref,
             out_ref, k_ref, v_ref, part_ref, rs_ref, ag_ref,
             rs_send, rs_recv, ag_send, ag_recv):
        my = lax.axis_index("i")
        left = lax.rem(my + (N_DEV - 1), N_DEV)
        right = lax.rem(my + 1, N_DEV)

        bar = pltpu.get_barrier_semaphore()
        for nbr in (left, right):
            pl.semaphore_signal(bar, inc=1, device_id=(nbr,),
                                device_id_type=pl.DeviceIdType.MESH)
        pl.semaphore_wait(bar, 2)

        def rope(t, cosr, sinr):
            n = t.shape[1]
            even = (lax.broadcasted_iota(jnp.int32, t.shape, 1) % 2) == 0
            t_next = pltpu.roll(t, n - 1, 1)
            t_prev = pltpu.roll(t, 1, 1)
            return t * cosr + jnp.where(even, -t_next, t_prev) * sinr

        xm = x_ref[0]
        cos_f = cos_ref[...]
        sin_f = sin_ref[...]

        k_ref[...] = rope(jnp.dot(xm, wk_ref[...],
                                  preferred_element_type=jnp.float32),
                          cos_f, sin_f).astype(jnp.bfloat16)
        v_ref[...] = jnp.dot(xm, wv_ref[...],
                             preferred_element_type=jnp.float32
                             ).astype(jnp.bfloat16)

        def compute_chunk(rc):
            ro = rc * CH
            xq = x_ref[0, pl.ds(ro, CH), :]
            q = rope(jnp.dot(xq, wq_ref[...],
                             preferred_element_type=jnp.float32),
                     cos_ref[pl.ds(ro, CH), :], sin_ref[pl.ds(ro, CH), :]
                     ).astype(jnp.bfloat16)
            outc = jnp.zeros((CH, D), jnp.float32)
            for h in range(HQ):
                sl = pl.ds(h * DH, DH)
                s = lax.dot_general(q[:, h * DH:(h + 1) * DH], k_ref[:, sl],
                                    (((1,), (1,)), ((), ())),
                                    preferred_element_type=jnp.float32) * SCALE
                m = jnp.max(s, axis=-1, keepdims=True)
                w = jnp.exp(s - m)
                w = (w / jnp.sum(w, axis=-1, keepdims=True)
                     ).astype(jnp.bfloat16)
                ctx = jnp.dot(w, v_ref[:, sl],
                              preferred_element_type=jnp.float32
                              ).astype(jnp.bfloat16)
                outc = outc + jnp.dot(ctx, wo_ref[sl, :],
                                      preferred_element_type=jnp.float32)
            return outc

        def rs_copy(src, dst_slot, step):
            return pltpu.make_async_remote_copy(
                src_ref=src, dst_ref=rs_ref.at[dst_slot],
                send_sem=rs_send.at[step], recv_sem=rs_recv.at[step],
                device_id=(right,), device_id_type=pl.DeviceIdType.MESH)

        part_ref[0] = compute_chunk(my)
        rs0 = rs_copy(part_ref.at[0], 0, 0)
        rs0.start()

        part_ref[1] = compute_chunk(lax.rem(my + 3, N_DEV))
        rs0.wait_recv()
        rs_ref[0] = rs_ref[0] + part_ref[1]
        rs1 = rs_copy(rs_ref.at[0], 1, 1)
        rs1.start()

        part_ref[2] = compute_chunk(lax.rem(my + 2, N_DEV))
        rs1.wait_recv()
        rs_ref[1] = rs_ref[1] + part_ref[2]
        rs2 = rs_copy(rs_ref.at[1], 2, 2)
        rs2.start()

        part_ref[3] = compute_chunk(lax.rem(my + 1, N_DEV))
        rs2.wait_recv()
        owned = rs_ref[2] + part_ref[3]
        part_ref[3] = owned
        out_ref[0, pl.ds(lax.rem(my + 1, N_DEV) * CH, CH), :] = owned

        def ag_copy(src, dst_slot, hop):
            return pltpu.make_async_remote_copy(
                src_ref=src, dst_ref=ag_ref.at[dst_slot],
                send_sem=ag_send.at[hop], recv_sem=ag_recv.at[hop],
                device_id=(right,), device_id_type=pl.DeviceIdType.MESH)

        ag0 = ag_copy(part_ref.at[3], 0, 0)
        ag0.start()
        ag0.wait_recv()
        ag1 = ag_copy(ag_ref.at[0], 1, 1)
        ag1.start()
        out_ref[0, pl.ds(my * CH, CH), :] = ag_ref[0]
        ag1.wait_recv()
        ag2 = ag_copy(ag_ref.at[1], 2, 2)
        ag2.start()
        out_ref[0, pl.ds(lax.rem(my + 3, N_DEV) * CH, CH), :] = ag_ref[1]
        ag2.wait_recv()
        out_ref[0, pl.ds(lax.rem(my + 2, N_DEV) * CH, CH), :] = ag_ref[2]

        for d in (rs0, rs1, rs2, ag0, ag1, ag2):
            d.wait_send()

    cos = jnp.asarray(_COS)
    sin = jnp.asarray(_SIN)
    return pl.pallas_call(
        body,
        out_shape=jax.ShapeDtypeStruct((1, SQ, D), jnp.float32),
        in_specs=[pl.BlockSpec(memory_space=pltpu.VMEM)] * 7,
        out_specs=pl.BlockSpec(memory_space=pltpu.VMEM),
        scratch_shapes=[
            pltpu.VMEM((SQ, D), jnp.bfloat16),
            pltpu.VMEM((SQ, D), jnp.bfloat16),
            pltpu.VMEM((N_DEV, CH, D), jnp.float32),
            pltpu.VMEM((N_DEV - 1, CH, D), jnp.float32),
            pltpu.VMEM((N_DEV - 1, CH, D), jnp.float32),
            pltpu.SemaphoreType.DMA((N_DEV - 1,)),
            pltpu.SemaphoreType.DMA((N_DEV - 1,)),
            pltpu.SemaphoreType.DMA((N_DEV - 1,)),
            pltpu.SemaphoreType.DMA((N_DEV - 1,)),
        ],
        compiler_params=pltpu.CompilerParams(
            collective_id=0, vmem_limit_bytes=100 * 1024 * 1024),
    )(x.astype(jnp.bfloat16), Wq.astype(jnp.bfloat16),
      Wk.astype(jnp.bfloat16), Wv.astype(jnp.bfloat16),
      Wo.astype(jnp.bfloat16), cos, sin)


# device time: 75877 ns/iter; 1.6015x vs baseline; 1.6015x over previous
import numpy as np
import jax
import jax.numpy as jnp
from jax import lax
from jax.experimental import pallas as pl
from jax.experimental.pallas import tpu as pltpu

N_DEV = 4
SQ = 1024
D = 1024
HQ = 8
DH = 128
CH = SQ // N_DEV
HD = D // 2
SCALE = 0.08838834764831843

_inv = 1.0 / (10000.0 ** (np.arange(0, DH, 2) / DH))
_pos = np.arange(SQ)[:, None] * _inv[None, :]
_COS = np.tile(np.repeat(np.cos(_pos), 2, axis=-1), (1, HQ)).astype(np.float32)
_SIN = np.tile(np.repeat(np.sin(_pos), 2, axis=-1), (1, HQ)).astype(np.float32)


def kernel(x, Wq, Wk, Wv, Wo):

    def body(x_ref, wq_ref, wk_ref, wv_ref, wo_ref, cos_ref, sin_ref,
             out_ref, k_ref, v_ref, ctx_ref,
             pr_ref, pl_ref, rsr_ref, rsl_ref, agr_ref, agl_ref,
             rsr_send, rsr_recv, rsl_send, rsl_recv,
             agr_send, agr_recv, agl_send, agl_recv):
        my = lax.axis_index("i")
        left = lax.rem(my + (N_DEV - 1), N_DEV)
        right = lax.rem(my + 1, N_DEV)

        bar = pltpu.get_barrier_semaphore()
        for nbr in (left, right):
            pl.semaphore_signal(bar, inc=1, device_id=(nbr,),
                                device_id_type=pl.DeviceIdType.MESH)
        pl.semaphore_wait(bar, 2)

        def rope(t, cosr, sinr):
            n = t.shape[1]
            even = (lax.broadcasted_iota(jnp.int32, t.shape, 1) % 2) == 0
            t_next = pltpu.roll(t, n - 1, 1)
            t_prev = pltpu.roll(t, 1, 1)
            return t * cosr + jnp.where(even, -t_next, t_prev) * sinr

        xm = x_ref[0]
        cos_f = cos_ref[...]
        sin_f = sin_ref[...]

        k_ref[...] = rope(jnp.dot(xm, wk_ref[...],
                                  preferred_element_type=jnp.float32),
                          cos_f, sin_f)
        v_ref[...] = jnp.dot(xm, wv_ref[...],
                             preferred_element_type=jnp.float32)

        def ctx_chunk(rc):
            ro = rc * CH
            xq = x_ref[0, pl.ds(ro, CH), :]
            q = rope(jnp.dot(xq, wq_ref[...],
                             preferred_element_type=jnp.float32),
                     cos_ref[pl.ds(ro, CH), :], sin_ref[pl.ds(ro, CH), :])
            parts = []
            for h in range(HQ):
                sl = pl.ds(h * DH, DH)
                s = lax.dot_general(q[:, h * DH:(h + 1) * DH], k_ref[:, sl],
                                    (((1,), (1,)), ((), ())),
                                    preferred_element_type=jnp.float32) * SCALE
                w = jnp.exp(s)
                w = w / jnp.sum(w, axis=-1, keepdims=True)
                parts.append(jnp.dot(w, v_ref[:, sl],
                                     preferred_element_type=jnp.float32))
            return jnp.concatenate(parts, axis=1)

        def proj_r(ctx):
            return jnp.dot(ctx, wo_ref[:, :HD],
                           preferred_element_type=jnp.float32)

        def proj_l(ctx):
            return jnp.dot(ctx, wo_ref[:, HD:],
                           preferred_element_type=jnp.float32)

        def copy(src, dst, send, recv, slot, dev):
            return pltpu.make_async_remote_copy(
                src_ref=src, dst_ref=dst.at[slot],
                send_sem=send.at[slot], recv_sem=recv.at[slot],
                device_id=(dev,), device_id_type=pl.DeviceIdType.MESH)

        ctx0 = ctx_chunk(my)
        pr_ref[0] = proj_r(ctx0)
        pl_ref[0] = proj_l(ctx0)
        rsr0 = copy(pr_ref.at[0], rsr_ref, rsr_send, rsr_recv, 0, right)
        rsr0.start()
        rsl0 = copy(pl_ref.at[0], rsl_ref, rsl_send, rsl_recv, 0, left)
        rsl0.start()

        ctx_ref[0] = ctx_chunk(lax.rem(my + 3, N_DEV))
        pr_ref[1] = proj_r(ctx_ref[0])
        ctx_ref[1] = ctx_chunk(lax.rem(my + 1, N_DEV))
        pl_ref[1] = proj_l(ctx_ref[1])
        rsr0.wait_recv()
        rsr_ref[0] = rsr_ref[0] + pr_ref[1]
        rsr1 = copy(rsr_ref.at[0], rsr_ref, rsr_send, rsr_recv, 1, right)
        rsr1.start()
        rsl0.wait_recv()
        rsl_ref[0] = rsl_ref[0] + pl_ref[1]
        rsl1 = copy(rsl_ref.at[0], rsl_ref, rsl_send, rsl_recv, 1, left)
        rsl1.start()

        ctx2 = ctx_chunk(lax.rem(my + 2, N_DEV))
        pr_ref[2] = proj_r(ctx2)
        pl_ref[2] = proj_l(ctx2)
        rsr1.wait_recv()
        rsr_ref[1] = rsr_ref[1] + pr_ref[2]
        rsr2 = copy(rsr_ref.at[1], rsr_ref, rsr_send, rsr_recv, 2, right)
        rsr2.start()
        rsl1.wait_recv()
        rsl_ref[1] = rsl_ref[1] + pl_ref[2]
        rsl2 = copy(rsl_ref.at[1], rsl_ref, rsl_send, rsl_recv, 2, left)
        rsl2.start()

        pr_ref[3] = proj_r(ctx_ref[1])
        pl_ref[3] = proj_l(ctx_ref[0])
        rsr2.wait_recv()
        owned_r = rsr_ref[2] + pr_ref[3]
        pr_ref[3] = owned_r
        out_ref[0, pl.ds(lax.rem(my + 1, N_DEV) * CH, CH), :HD] = owned_r
        rsl2.wait_recv()
        owned_l = rsl_ref[2] + pl_ref[3]
        pl_ref[3] = owned_l
        out_ref[0, pl.ds(lax.rem(my + 3, N_DEV) * CH, CH), HD:] = owned_l

        agr0 = copy(pr_ref.at[3], agr_ref, agr_send, agr_recv, 0, right)
        agr0.start()
        agl0 = copy(pl_ref.at[3], agl_ref, agl_send, agl_recv, 0, left)
        agl0.start()

        agr0.wait_recv()
        agr1 = copy(agr_ref.at[0], agr_ref, agr_send, agr_recv, 1, right)
        agr1.start()
        out_ref[0, pl.ds(my * CH, CH), :HD] = agr_ref[0]
        agl0.wait_recv()
        agl1 = copy(agl_ref.at[0], agl_ref, agl_send, agl_recv, 1, left)
        agl1.start()
        out_ref[0, pl.ds(my * CH, CH), HD:] = agl_ref[0]

        agr1.wait_recv()
        agr2 = copy(agr_ref.at[1], agr_ref, agr_send, agr_recv, 2, right)
        agr2.start()
        out_ref[0, pl.ds(lax.rem(my + 3, N_DEV) * CH, CH), :HD] = agr_ref[1]
        agl1.wait_recv()
        agl2 = copy(agl_ref.at[1], agl_ref, agl_send, agl_recv, 2, left)
        agl2.start()
        out_ref[0, pl.ds(lax.rem(my + 1, N_DEV) * CH, CH), HD:] = agl_ref[1]

        agr2.wait_recv()
        out_ref[0, pl.ds(lax.rem(my + 2, N_DEV) * CH, CH), :HD] = agr_ref[2]
        agl2.wait_recv()
        out_ref[0, pl.ds(lax.rem(my + 2, N_DEV) * CH, CH), HD:] = agl_ref[2]

        for d in (rsr0, rsr1, rsr2, rsl0, rsl1, rsl2,
                  agr0, agr1, agr2, agl0, agl1, agl2):
            d.wait_send()

    cos = jnp.asarray(_COS)
    sin = jnp.asarray(_SIN)
    dma3 = pltpu.SemaphoreType.DMA((N_DEV - 1,))
    return pl.pallas_call(
        body,
        out_shape=jax.ShapeDtypeStruct((1, SQ, D), jnp.float32),
        in_specs=[pl.BlockSpec(memory_space=pltpu.VMEM)] * 7,
        out_specs=pl.BlockSpec(memory_space=pltpu.VMEM),
        scratch_shapes=[
            pltpu.VMEM((SQ, D), jnp.float32),
            pltpu.VMEM((SQ, D), jnp.float32),
            pltpu.VMEM((2, CH, D), jnp.float32),
            pltpu.VMEM((N_DEV, CH, HD), jnp.float32),
            pltpu.VMEM((N_DEV, CH, HD), jnp.float32),
            pltpu.VMEM((N_DEV - 1, CH, HD), jnp.float32),
            pltpu.VMEM((N_DEV - 1, CH, HD), jnp.float32),
            pltpu.VMEM((N_DEV - 1, CH, HD), jnp.float32),
            pltpu.VMEM((N_DEV - 1, CH, HD), jnp.float32),
            dma3, dma3, dma3, dma3,
            dma3, dma3, dma3, dma3,
        ],
        compiler_params=pltpu.CompilerParams(
            collective_id=0, vmem_limit_bytes=100 * 1024 * 1024),
    )(x, Wq, Wk, Wv, Wo, cos, sin)


# device time: 73573 ns/iter; 1.6517x vs baseline; 1.0313x over previous
import numpy as np
import jax
import jax.numpy as jnp
from jax import lax
from jax.experimental import pallas as pl
from jax.experimental.pallas import tpu as pltpu

N_DEV = 4
SQ = 1024
D = 1024
HQ = 8
DH = 128
CH = SQ // N_DEV
HD = D // 2
SCALE = 0.08838834764831843

_inv = 1.0 / (10000.0 ** (np.arange(0, DH, 2) / DH))
_pos = np.arange(SQ)[:, None] * _inv[None, :]
_COS = np.tile(np.repeat(np.cos(_pos), 2, axis=-1), (1, HQ)).astype(np.float32)
_SIN = np.tile(np.repeat(np.sin(_pos), 2, axis=-1), (1, HQ)).astype(np.float32)


def kernel(x, Wq, Wk, Wv, Wo):

    def body(x_ref, wq_ref, wk_ref, wv_ref, wo_ref, cos_ref, sin_ref,
             out_ref, k_ref, v_ref, ctx_ref,
             pr_ref, pl_ref, rsr_ref, rsl_ref, agr_ref, agl_ref,
             rsr_send, rsr_recv, rsl_send, rsl_recv,
             agr_send, agr_recv, agl_send, agl_recv):
        my = lax.axis_index("i")
        left = lax.rem(my + (N_DEV - 1), N_DEV)
        right = lax.rem(my + 1, N_DEV)

        bar = pltpu.get_barrier_semaphore()
        for nbr in (left, right):
            pl.semaphore_signal(bar, inc=1, device_id=(nbr,),
                                device_id_type=pl.DeviceIdType.MESH)
        pl.semaphore_wait(bar, 2)

        def rope(t, cosr, sinr):
            n = t.shape[1]
            even = (lax.broadcasted_iota(jnp.int32, t.shape, 1) % 2) == 0
            t_next = pltpu.roll(t, n - 1, 1)
            t_prev = pltpu.roll(t, 1, 1)
            return t * cosr + jnp.where(even, -t_next, t_prev) * sinr

        xm = x_ref[0]
        cos_f = cos_ref[...]
        sin_f = sin_ref[...]

        k_ref[...] = rope(jnp.dot(xm, wk_ref[...],
                                  preferred_element_type=jnp.float32),
                          cos_f, sin_f)
        v_ref[...] = jnp.dot(xm, wv_ref[...],
                             preferred_element_type=jnp.float32)

        def ctx_chunk(rc):
            ro = rc * CH
            xq = x_ref[0, pl.ds(ro, CH), :]
            q = rope(jnp.dot(xq, wq_ref[...],
                             preferred_element_type=jnp.float32),
                     cos_ref[pl.ds(ro, CH), :], sin_ref[pl.ds(ro, CH), :])
            q = q * SCALE
            parts = []
            for h in range(HQ):
                sl = pl.ds(h * DH, DH)
                s = lax.dot_general(q[:, h * DH:(h + 1) * DH], k_ref[:, sl],
                                    (((1,), (1,)), ((), ())),
                                    preferred_element_type=jnp.float32)
                w = jnp.exp(s)
                ctx = jnp.dot(w, v_ref[:, sl],
                              preferred_element_type=jnp.float32)
                parts.append(ctx / jnp.sum(w, axis=-1, keepdims=True))
            return jnp.concatenate(parts, axis=1)

        def proj_r(ctx):
            return jnp.dot(ctx, wo_ref[:, :HD],
                           preferred_element_type=jnp.float32)

        def proj_l(ctx):
            return jnp.dot(ctx, wo_ref[:, HD:],
                           preferred_element_type=jnp.float32)

        def copy(src, dst, send, recv, slot, dev):
            return pltpu.make_async_remote_copy(
                src_ref=src, dst_ref=dst.at[slot],
                send_sem=send.at[slot], recv_sem=recv.at[slot],
                device_id=(dev,), device_id_type=pl.DeviceIdType.MESH)

        ctx0 = ctx_chunk(my)
        pr_ref[0] = proj_r(ctx0)
        pl_ref[0] = proj_l(ctx0)
        rsr0 = copy(pr_ref.at[0], rsr_ref, rsr_send, rsr_recv, 0, right)
        rsr0.start()
        rsl0 = copy(pl_ref.at[0], rsl_ref, rsl_send, rsl_recv, 0, left)
        rsl0.start()

        ctx_ref[0] = ctx_chunk(lax.rem(my + 3, N_DEV))
        pr_ref[1] = proj_r(ctx_ref[0])
        ctx_ref[1] = ctx_chunk(lax.rem(my + 1, N_DEV))
        pl_ref[1] = proj_l(ctx_ref[1])
        rsr0.wait_recv()
        rsr_ref[0] = rsr_ref[0] + pr_ref[1]
        rsr1 = copy(rsr_ref.at[0], rsr_ref, rsr_send, rsr_recv, 1, right)
        rsr1.start()
        rsl0.wait_recv()
        rsl_ref[0] = rsl_ref[0] + pl_ref[1]
        rsl1 = copy(rsl_ref.at[0], rsl_ref, rsl_send, rsl_recv, 1, left)
        rsl1.start()

        ctx2 = ctx_chunk(lax.rem(my + 2, N_DEV))
        pr_ref[2] = proj_r(ctx2)
        pl_ref[2] = proj_l(ctx2)
        rsr1.wait_recv()
        rsr_ref[1] = rsr_ref[1] + pr_ref[2]
        rsr2 = copy(rsr_ref.at[1], rsr_ref, rsr_send, rsr_recv, 2, right)
        rsr2.start()
        rsl1.wait_recv()
        rsl_ref[1] = rsl_ref[1] + pl_ref[2]
        rsl2 = copy(rsl_ref.at[1], rsl_ref, rsl_send, rsl_recv, 2, left)
        rsl2.start()

        pr_ref[3] = proj_r(ctx_ref[1])
        pl_ref[3] = proj_l(ctx_ref[0])
        rsr2.wait_recv()
        pr_ref[3] = rsr_ref[2] + pr_ref[3]
        agr0 = copy(pr_ref.at[3], agr_ref, agr_send, agr_recv, 0, right)
        agr0.start()
        rsl2.wait_recv()
        pl_ref[3] = rsl_ref[2] + pl_ref[3]
        agl0 = copy(pl_ref.at[3], agl_ref, agl_send, agl_recv, 0, left)
        agl0.start()
        out_ref[0, pl.ds(lax.rem(my + 1, N_DEV) * CH, CH), :HD] = pr_ref[3]
        out_ref[0, pl.ds(lax.rem(my + 3, N_DEV) * CH, CH), HD:] = pl_ref[3]

        agr0.wait_recv()
        agr1 = copy(agr_ref.at[0], agr_ref, agr_send, agr_recv, 1, right)
        agr1.start()
        out_ref[0, pl.ds(my * CH, CH), :HD] = agr_ref[0]
        agl0.wait_recv()
        agl1 = copy(agl_ref.at[0], agl_ref, agl_send, agl_recv, 1, left)
        agl1.start()
        out_ref[0, pl.ds(my * CH, CH), HD:] = agl_ref[0]

        agr1.wait_recv()
        agr2 = copy(agr_ref.at[1], agr_ref, agr_send, agr_recv, 2, right)
        agr2.start()
        out_ref[0, pl.ds(lax.rem(my + 3, N_DEV) * CH, CH), :HD] = agr_ref[1]
        agl1.wait_recv()
        agl2 = copy(agl_ref.at[1], agl_ref, agl_send, agl_recv, 2, left)
        agl2.start()
        out_ref[0, pl.ds(lax.rem(my + 1, N_DEV) * CH, CH), HD:] = agl_ref[1]

        agr2.wait_recv()
        out_ref[0, pl.ds(lax.rem(my + 2, N_DEV) * CH, CH), :HD] = agr_ref[2]
        agl2.wait_recv()
        out_ref[0, pl.ds(lax.rem(my + 2, N_DEV) * CH, CH), HD:] = agl_ref[2]

        for d in (rsr0, rsr1, rsr2, rsl0, rsl1, rsl2,
                  agr0, agr1, agr2, agl0, agl1, agl2):
            d.wait_send()

    cos = jnp.asarray(_COS)
    sin = jnp.asarray(_SIN)
    dma3 = pltpu.SemaphoreType.DMA((N_DEV - 1,))
    return pl.pallas_call(
        body,
        out_shape=jax.ShapeDtypeStruct((1, SQ, D), jnp.float32),
        in_specs=[pl.BlockSpec(memory_space=pltpu.VMEM)] * 7,
        out_specs=pl.BlockSpec(memory_space=pltpu.VMEM),
        scratch_shapes=[
            pltpu.VMEM((SQ, D), jnp.float32),
            pltpu.VMEM((SQ, D), jnp.float32),
            pltpu.VMEM((2, CH, D), jnp.float32),
            pltpu.VMEM((N_DEV, CH, HD), jnp.float32),
            pltpu.VMEM((N_DEV, CH, HD), jnp.float32),
            pltpu.VMEM((N_DEV - 1, CH, HD), jnp.float32),
            pltpu.VMEM((N_DEV - 1, CH, HD), jnp.float32),
            pltpu.VMEM((N_DEV - 1, CH, HD), jnp.float32),
            pltpu.VMEM((N_DEV - 1, CH, HD), jnp.float32),
            dma3, dma3, dma3, dma3,
            dma3, dma3, dma3, dma3,
        ],
        compiler_params=pltpu.CompilerParams(
            collective_id=0, vmem_limit_bytes=100 * 1024 * 1024),
    )(x, Wq, Wk, Wv, Wo, cos, sin)


# device time: 59383 ns/iter; 2.0464x vs baseline; 1.2390x over previous
import numpy as np
import jax
import jax.numpy as jnp
from jax import lax
from jax.experimental import pallas as pl
from jax.experimental.pallas import tpu as pltpu

N_DEV = 4
SQ = 1024
D = 1024
HQ = 8
DH = 128
CH = SQ // N_DEV
HD = D // 2
SCALE = 0.08838834764831843

_inv = 1.0 / (10000.0 ** (np.arange(0, DH, 2) / DH))
_pos = np.arange(SQ)[:, None] * _inv[None, :]
_COS = np.tile(np.repeat(np.cos(_pos), 2, axis=-1), (1, HQ)).astype(np.float32)
_SIN = np.tile(np.repeat(np.sin(_pos), 2, axis=-1), (1, HQ)).astype(np.float32)


def kernel(x, Wq, Wk, Wv, Wo):

    def body(x_ref, wq_ref, wk_ref, wv_ref, wo_ref, cos_ref, sin_ref,
             out_ref, k_ref, v_ref, ctx_ref,
             pr_ref, pl_ref, rsr_ref, rsl_ref, agr_ref, agl_ref,
             rsr_send, rsr_recv, rsl_send, rsl_recv,
             agr_send, agr_recv, agl_send, agl_recv):
        my = lax.axis_index("i")
        left = lax.rem(my + (N_DEV - 1), N_DEV)
        right = lax.rem(my + 1, N_DEV)

        bar = pltpu.get_barrier_semaphore()
        for nbr in (left, right):
            pl.semaphore_signal(bar, inc=1, device_id=(nbr,),
                                device_id_type=pl.DeviceIdType.MESH)
        pl.semaphore_wait(bar, 2)

        def rope(t, cosr, sinr):
            n = t.shape[1]
            even = (lax.broadcasted_iota(jnp.int32, t.shape, 1) % 2) == 0
            t_next = pltpu.roll(t, n - 1, 1)
            t_prev = pltpu.roll(t, 1, 1)
            return t * cosr + jnp.where(even, -t_next, t_prev) * sinr

        xm = x_ref[0]
        cos_f = cos_ref[...]
        sin_f = sin_ref[...]

        k_ref[...] = rope(jnp.dot(xm, wk_ref[...],
                                  preferred_element_type=jnp.float32),
                          cos_f, sin_f)
        v_ref[...] = jnp.dot(xm, wv_ref[...],
                             preferred_element_type=jnp.float32)

        def ctx_chunk(rc):
            ro = rc * CH
            xq = x_ref[0, pl.ds(ro, CH), :]
            q = rope(jnp.dot(xq, wq_ref[...],
                             preferred_element_type=jnp.float32),
                     cos_ref[pl.ds(ro, CH), :], sin_ref[pl.ds(ro, CH), :])
            q = q * SCALE
            parts = []
            for h in range(HQ):
                sl = pl.ds(h * DH, DH)
                s = lax.dot_general(q[:, h * DH:(h + 1) * DH], k_ref[:, sl],
                                    (((1,), (1,)), ((), ())),
                                    preferred_element_type=jnp.float32)
                w = jnp.exp(s)
                ctx = jnp.dot(w, v_ref[:, sl],
                              preferred_element_type=jnp.float32)
                parts.append(ctx / jnp.sum(w, axis=-1, keepdims=True))
            return jnp.concatenate(parts, axis=1)

        def proj_r(ctx):
            return jnp.dot(ctx, wo_ref[:, :HD],
                           preferred_element_type=jnp.float32
                           ).astype(jnp.bfloat16)

        def proj_l(ctx):
            return jnp.dot(ctx, wo_ref[:, HD:],
                           preferred_element_type=jnp.float32
                           ).astype(jnp.bfloat16)

        def add_bf(a, b):
            return (a.astype(jnp.float32) + b.astype(jnp.float32)
                    ).astype(jnp.bfloat16)

        def copy(src, dst, send, recv, slot, dev):
            return pltpu.make_async_remote_copy(
                src_ref=src, dst_ref=dst.at[slot],
                send_sem=send.at[slot], recv_sem=recv.at[slot],
                device_id=(dev,), device_id_type=pl.DeviceIdType.MESH)

        ctx0 = ctx_chunk(my)
        pr_ref[0] = proj_r(ctx0)
        pl_ref[0] = proj_l(ctx0)
        rsr0 = copy(pr_ref.at[0], rsr_ref, rsr_send, rsr_recv, 0, right)
        rsr0.start()
        rsl0 = copy(pl_ref.at[0], rsl_ref, rsl_send, rsl_recv, 0, left)
        rsl0.start()

        ctx_ref[0] = ctx_chunk(lax.rem(my + 3, N_DEV))
        pr_ref[1] = proj_r(ctx_ref[0])
        ctx_ref[1] = ctx_chunk(lax.rem(my + 1, N_DEV))
        pl_ref[1] = proj_l(ctx_ref[1])
        rsr0.wait_recv()
        rsr_ref[0] = add_bf(rsr_ref[0], pr_ref[1])
        rsr1 = copy(rsr_ref.at[0], rsr_ref, rsr_send, rsr_recv, 1, right)
        rsr1.start()
        rsl0.wait_recv()
        rsl_ref[0] = add_bf(rsl_ref[0], pl_ref[1])
        rsl1 = copy(rsl_ref.at[0], rsl_ref, rsl_send, rsl_recv, 1, left)
        rsl1.start()

        ctx2 = ctx_chunk(lax.rem(my + 2, N_DEV))
        pr_ref[2] = proj_r(ctx2)
        pl_ref[2] = proj_l(ctx2)
        rsr1.wait_recv()
        rsr_ref[1] = add_bf(rsr_ref[1], pr_ref[2])
        rsr2 = copy(rsr_ref.at[1], rsr_ref, rsr_send, rsr_recv, 2, right)
        rsr2.start()
        rsl1.wait_recv()
        rsl_ref[1] = add_bf(rsl_ref[1], pl_ref[2])
        rsl2 = copy(rsl_ref.at[1], rsl_ref, rsl_send, rsl_recv, 2, left)
        rsl2.start()

        pr_ref[3] = proj_r(ctx_ref[1])
        pl_ref[3] = proj_l(ctx_ref[0])
        rsr2.wait_recv()
        pr_ref[3] = add_bf(rsr_ref[2], pr_ref[3])
        agr0 = copy(pr_ref.at[3], agr_ref, agr_send, agr_recv, 0, right)
        agr0.start()
        rsl2.wait_recv()
        pl_ref[3] = add_bf(rsl_ref[2], pl_ref[3])
        agl0 = copy(pl_ref.at[3], agl_ref, agl_send, agl_recv, 0, left)
        agl0.start()
        out_ref[0, pl.ds(lax.rem(my + 1, N_DEV) * CH, CH), :HD] = (
            pr_ref[3].astype(jnp.float32))
        out_ref[0, pl.ds(lax.rem(my + 3, N_DEV) * CH, CH), HD:] = (
            pl_ref[3].astype(jnp.float32))

        agr0.wait_recv()
        agr1 = copy(agr_ref.at[0], agr_ref, agr_send, agr_recv, 1, right)
        agr1.start()
        out_ref[0, pl.ds(my * CH, CH), :HD] = agr_ref[0].astype(jnp.float32)
        agl0.wait_recv()
        agl1 = copy(agl_ref.at[0], agl_ref, agl_send, agl_recv, 1, left)
        agl1.start()
        out_ref[0, pl.ds(my * CH, CH), HD:] = agl_ref[0].astype(jnp.float32)

        agr1.wait_recv()
        agr2 = copy(agr_ref.at[1], agr_ref, agr_send, agr_recv, 2, right)
        agr2.start()
        out_ref[0, pl.ds(lax.rem(my + 3, N_DEV) * CH, CH), :HD] = (
            agr_ref[1].astype(jnp.float32))
        agl1.wait_recv()
        agl2 = copy(agl_ref.at[1], agl_ref, agl_send, agl_recv, 2, left)
        agl2.start()
        out_ref[0, pl.ds(lax.rem(my + 1, N_DEV) * CH, CH), HD:] = (
            agl_ref[1].astype(jnp.float32))

        agr2.wait_recv()
        out_ref[0, pl.ds(lax.rem(my + 2, N_DEV) * CH, CH), :HD] = (
            agr_ref[2].astype(jnp.float32))
        agl2.wait_recv()
        out_ref[0, pl.ds(lax.rem(my + 2, N_DEV) * CH, CH), HD:] = (
            agl_ref[2].astype(jnp.float32))

        for d in (rsr0, rsr1, rsr2, rsl0, rsl1, rsl2,
                  agr0, agr1, agr2, agl0, agl1, agl2):
            d.wait_send()

    cos = jnp.asarray(_COS)
    sin = jnp.asarray(_SIN)
    dma3 = pltpu.SemaphoreType.DMA((N_DEV - 1,))
    return pl.pallas_call(
        body,
        out_shape=jax.ShapeDtypeStruct((1, SQ, D), jnp.float32),
        in_specs=[pl.BlockSpec(memory_space=pltpu.VMEM)] * 7,
        out_specs=pl.BlockSpec(memory_space=pltpu.VMEM),
        scratch_shapes=[
            pltpu.VMEM((SQ, D), jnp.float32),
            pltpu.VMEM((SQ, D), jnp.float32),
            pltpu.VMEM((2, CH, D), jnp.float32),
            pltpu.VMEM((N_DEV, CH, HD), jnp.bfloat16),
            pltpu.VMEM((N_DEV, CH, HD), jnp.bfloat16),
            pltpu.VMEM((N_DEV - 1, CH, HD), jnp.bfloat16),
            pltpu.VMEM((N_DEV - 1, CH, HD), jnp.bfloat16),
            pltpu.VMEM((N_DEV - 1, CH, HD), jnp.bfloat16),
            pltpu.VMEM((N_DEV - 1, CH, HD), jnp.bfloat16),
            dma3, dma3, dma3, dma3,
            dma3, dma3, dma3, dma3,
        ],
        compiler_params=pltpu.CompilerParams(
            collective_id=0, vmem_limit_bytes=100 * 1024 * 1024),
    )(x, Wq, Wk, Wv, Wo, cos, sin)
